# baseline (speedup 1.0000x reference)
import functools

import jax
import jax.numpy as jnp
from jax import lax
from jax.experimental import pallas as pl
from jax.experimental.pallas import tpu as pltpu
from jax.experimental.pallas import tpu_sc as plsc

B = 16384
EMB = 64
NC = 2
NS = 16
L = 16
NW = NC * NS
BPW = B // NW


def _sc_tanh(v):
    return 1.0 - 2.0 / (jnp.exp(v * 2.0) + 1.0)


def _scores_body(hidx_hbm, ridx_hbm, tidx_hbm, ent_hbm, rel_hbm, out_hbm,
                 hidx_v, ridx_v, tidx_v, hrows, rrows, trows, sc_v, sem):
    wid = lax.axis_index("s") * NC + lax.axis_index("c")
    base = wid * BPW

    pltpu.sync_copy(hidx_hbm.at[pl.ds(base, BPW)], hidx_v)
    pltpu.sync_copy(ridx_hbm.at[pl.ds(base, BPW)], ridx_v)
    pltpu.sync_copy(tidx_hbm.at[pl.ds(base, BPW)], tidx_v)

    ch = pltpu.make_async_copy(ent_hbm.at[hidx_v], hrows, sem)
    cr = pltpu.make_async_copy(rel_hbm.at[ridx_v], rrows, sem)
    ct = pltpu.make_async_copy(ent_hbm.at[tidx_v], trows, sem)
    ch.start()
    cr.start()
    ct.start()
    ch.wait()
    cr.wait()
    ct.wait()

    lanes = lax.iota(jnp.int32, L)

    def group_body(g, carry):
        row0 = g * L

        def row_body(k, svec):
            r = row0 + k
            acc = jnp.zeros((L,), jnp.float32)
            for c in range(EMB // L):
                hv = hrows[r, pl.ds(c * L, L)]
                rv = rrows[r, pl.ds(c * L, L)]
                tv = trows[r, pl.ds(c * L, L)]
                acc = acc + _sc_tanh(hv) * _sc_tanh(rv) * _sc_tanh(tv)
            s = jnp.sum(acc)
            return jnp.where(lanes == k, s, svec)

        svec = lax.fori_loop(0, L, row_body, jnp.zeros((L,), jnp.float32))
        sc_v[pl.ds(row0, L)] = svec
        return carry

    lax.fori_loop(0, BPW // L, group_body, 0)
    pltpu.sync_copy(sc_v, out_hbm.at[pl.ds(base, BPW)])


def _sc_scores(h_idx, r_idx, t_idx, entity_emb, relation_emb):
    mesh = plsc.VectorSubcoreMesh(core_axis_name="c", subcore_axis_name="s")
    run = functools.partial(
        pl.kernel,
        mesh=mesh,
        compiler_params=pltpu.CompilerParams(
            needs_layout_passes=False, use_tc_tiling_on_sc=False
        ),
        out_type=jax.ShapeDtypeStruct((B,), jnp.float32),
        scratch_types=[
            pltpu.VMEM((BPW,), jnp.int32),
            pltpu.VMEM((BPW,), jnp.int32),
            pltpu.VMEM((BPW,), jnp.int32),
            pltpu.VMEM((BPW, EMB), jnp.float32),
            pltpu.VMEM((BPW, EMB), jnp.float32),
            pltpu.VMEM((BPW, EMB), jnp.float32),
            pltpu.VMEM((BPW,), jnp.float32),
            pltpu.SemaphoreType.DMA,
        ],
    )(_scores_body)
    return run(h_idx, r_idx, t_idx, entity_emb, relation_emb)


def _regul_body(s_ref, o_ref):
    o_ref[0, 0] = jnp.abs(jnp.sum(s_ref[...]))


def _tc_regul(scores2d):
    out = pl.pallas_call(
        _regul_body,
        out_shape=jax.ShapeDtypeStruct((1, 1), jnp.float32),
        out_specs=pl.BlockSpec(memory_space=pltpu.SMEM),
    )(scores2d)
    return out[0, 0]


def kernel(x, entity_emb, relation_emb):
    h_idx = x[:, 0]
    r_idx = x[:, 1]
    t_idx = x[:, 2]
    scores = _sc_scores(h_idx, r_idx, t_idx, entity_emb, relation_emb)
    regul = _tc_regul(scores.reshape(B // 128, 128))
    return (scores, regul)
